# baseline (speedup 1.0000x reference)
import functools

import jax
import jax.numpy as jnp
from jax import lax
from jax.experimental import pallas as pl
from jax.experimental.pallas import tpu as pltpu
from jax.experimental.pallas import tpu_sc as plsc

_SCALE = 8.0
_LANES = 16


@functools.cache
def _make_gather(V, D, N):
    info = plsc.get_sparse_core_info()
    NC, NS = info.num_cores, info.num_subcores
    NW = NC * NS
    assert N % NW == 0
    RW = N // NW
    CH = 512
    assert RW % CH == 0
    NCHUNK = RW // CH

    mesh = plsc.VectorSubcoreMesh(core_axis_name="c", subcore_axis_name="s")

    @functools.partial(
        pl.kernel,
        mesh=mesh,
        out_type=jax.ShapeDtypeStruct((N, D), jnp.float32),
        scratch_types=[
            pltpu.VMEM((CH,), jnp.int32),
            pltpu.VMEM((CH, D), jnp.float32),
            pltpu.SemaphoreType.DMA,
        ],
        compiler_params=pltpu.CompilerParams(use_tc_tiling_on_sc=False),
    )
    def gather_kernel(table_hbm, idx_hbm, out_hbm, idx_v, rows_v, sem):
        wid = lax.axis_index("s") * NC + lax.axis_index("c")
        base = wid * RW

        def chunk_body(c, carry):
            row0 = base + c * CH
            pltpu.sync_copy(idx_hbm.at[pl.ds(row0, CH)], idx_v)
            pltpu.async_copy(table_hbm.at[idx_v], rows_v, sem).wait()

            def scale_body(r, inner):
                for d in range(D // _LANES):
                    sl = (r, pl.ds(d * _LANES, _LANES))
                    rows_v[sl] = rows_v[sl] * _SCALE
                return inner

            lax.fori_loop(0, CH, scale_body, 0)
            pltpu.sync_copy(rows_v, out_hbm.at[pl.ds(row0, CH)])
            return carry

        lax.fori_loop(0, NCHUNK, chunk_body, 0)

    return gather_kernel


def kernel(x, table):
    B, H = x.shape
    V, D = table.shape
    N = B * H
    flat = x.reshape(N).astype(jnp.int32)
    out = _make_gather(V, D, N)(table, flat)
    return out.reshape(B, H, D)


# speedup vs baseline: 1.1399x; 1.1399x over previous
import functools

import jax
import jax.numpy as jnp
from jax import lax
from jax.experimental import pallas as pl
from jax.experimental.pallas import tpu as pltpu
from jax.experimental.pallas import tpu_sc as plsc

_SCALE = 8.0
_LANES = 16


@functools.cache
def _make_gather(V, D, N):
    info = plsc.get_sparse_core_info()
    NC, NS = info.num_cores, info.num_subcores
    NW = NC * NS
    assert N % NW == 0
    RW = N // NW
    CH = 512
    assert RW % (2 * CH) == 0
    NCHUNK = RW // CH

    mesh = plsc.VectorSubcoreMesh(core_axis_name="c", subcore_axis_name="s")

    @functools.partial(
        pl.kernel,
        mesh=mesh,
        out_type=jax.ShapeDtypeStruct((N, D), jnp.float32),
        scratch_types=[
            pltpu.VMEM((RW,), jnp.int32),
            pltpu.VMEM((CH, D), jnp.float32),
            pltpu.VMEM((CH, D), jnp.float32),
            pltpu.SemaphoreType.DMA,
            pltpu.SemaphoreType.DMA,
            pltpu.SemaphoreType.DMA,
            pltpu.SemaphoreType.DMA,
        ],
        compiler_params=pltpu.CompilerParams(use_tc_tiling_on_sc=False),
    )
    def gather_kernel(table_hbm, idx_hbm, out_hbm,
                      idx_v, rows0, rows1, gs0, gs1, os0, os1):
        rows = (rows0, rows1)
        gsem = (gs0, gs1)
        osem = (os0, os1)
        wid = lax.axis_index("s") * NC + lax.axis_index("c")
        base = wid * RW

        pltpu.sync_copy(idx_hbm.at[pl.ds(base, RW)], idx_v)

        def start_gather(c, b):
            pltpu.async_copy(
                table_hbm.at[idx_v.at[pl.ds(c * CH, CH)]], rows[b], gsem[b])

        def wait_gather(c, b):
            pltpu.make_async_copy(
                table_hbm.at[idx_v.at[pl.ds(c * CH, CH)]], rows[b],
                gsem[b]).wait()

        def start_out(c, b):
            pltpu.async_copy(
                rows[b], out_hbm.at[pl.ds(base + c * CH, CH)], osem[b])

        def wait_out(c, b):
            pltpu.make_async_copy(
                rows[b], out_hbm.at[pl.ds(base + c * CH, CH)], osem[b]).wait()

        start_gather(0, 0)
        start_gather(1, 1)

        def pair_body(p, carry):
            for b in range(2):
                c = p * 2 + b
                wait_gather(c, b)
                buf = rows[b]

                @plsc.parallel_loop(0, CH, unroll=4)
                def _(r):
                    for d in range(D // _LANES):
                        sl = (r, pl.ds(d * _LANES, _LANES))
                        buf[sl] = buf[sl] * _SCALE

                start_out(c, b)

                @pl.when(c + 2 < NCHUNK)
                def _():
                    wait_out(c, b)
                    start_gather(c + 2, b)

            return carry

        lax.fori_loop(0, NCHUNK // 2, pair_body, 0)
        wait_out(NCHUNK - 2, 0)
        wait_out(NCHUNK - 1, 1)

    return gather_kernel


def kernel(x, table):
    B, H = x.shape
    V, D = table.shape
    N = B * H
    flat = x.reshape(N).astype(jnp.int32)
    out = _make_gather(V, D, N)(table, flat)
    return out.reshape(B, H, D)
